# baseline (speedup 1.0000x reference)
import jax
import jax.numpy as jnp
from jax.experimental import pallas as pl
from jax.experimental.pallas import tpu as pltpu


_N = 2097152
_NCH = 128
_CH = _N // _NCH
_CF = _CH // 16


def _body(tok_ref, wt_ref, b_ref, x_ref, o_ref):
    wt = wt_ref[...]
    bb = b_ref[...]
    row0 = jnp.dot(tok_ref[...], wt,
                   preferred_element_type=jnp.float32) + bb
    y1 = jnp.dot(x_ref[...], wt, preferred_element_type=jnp.float32) + bb
    o_ref[0] = jnp.broadcast_to(row0, (_CH, 16))
    o_ref[1] = y1


def kernel(x, token, W, b):
    wt = W.T
    b2 = b.reshape(1, 16)
    tok2 = token.reshape(1, 8)
    call = pl.pallas_call(
        _body,
        in_specs=[
            pl.BlockSpec(memory_space=pltpu.MemorySpace.VMEM),
            pl.BlockSpec(memory_space=pltpu.MemorySpace.VMEM),
            pl.BlockSpec(memory_space=pltpu.MemorySpace.VMEM),
            pl.BlockSpec(memory_space=pltpu.MemorySpace.VMEM),
        ],
        out_specs=pl.BlockSpec(memory_space=pltpu.MemorySpace.VMEM),
        out_shape=jax.ShapeDtypeStruct((2, _CH, 16), jnp.float32),
    )
    outs = []
    for i in range(_NCH):
        xc = jax.lax.dynamic_slice(x, (1, i * _CH, 0), (1, _CH, 8))[0]
        outs.append(call(tok2, wt, b2, xc))
    return jnp.concatenate(outs, axis=1)
